# baseline (speedup 1.0000x reference)
import functools

import jax
import jax.numpy as jnp
from jax.experimental import pallas as pl
from jax.experimental.pallas import tpu as pltpu

_D_MODEL = 4096
_NUM_EXPERTS = 64
_TOP_K = 8
_BT = 256
_NS = 4


def _gate_half(x_blk, w):
    logits_t = jax.lax.dot_general(
        w, x_blk,
        dimension_numbers=(((1,), (1,)), ((), ())),
        preferred_element_type=jnp.float32,
    )
    m = jnp.max(logits_t, axis=0, keepdims=True)
    e = jnp.exp(logits_t - m)
    s = jnp.sum(e, axis=0, keepdims=True)
    probs_t = e / s
    psum = jnp.sum(probs_t, axis=0, keepdims=True)

    eiota = jax.lax.broadcasted_iota(jnp.int32, probs_t.shape, 0).astype(
        jnp.float32
    )
    cur = probs_t
    mxs = []
    idxs = []
    for k in range(_TOP_K):
        mx = jnp.max(cur, axis=0, keepdims=True)
        idxf = jnp.min(
            jnp.where(cur == mx, eiota, float(_NUM_EXPERTS)),
            axis=0, keepdims=True,
        )
        mxs.append(mx / psum)
        idxs.append(idxf)
        if k + 1 < _TOP_K:
            cur = jnp.where(eiota == idxf, -1.0, cur)
    topk_w_t = jnp.concatenate(mxs, axis=0)
    topk_i_t = jnp.concatenate(idxs, axis=0)
    return (topk_w_t.T, probs_t.T, topk_i_t.T.astype(jnp.int32), logits_t.T)


def _gate_kernel(x1_ref, x2_ref, x3_ref, x4_ref, w_ref, topk_w_ref,
                 probs_ref, topk_i_ref, logits_ref):
    w = w_ref[...]
    for h, x_ref in enumerate((x1_ref, x2_ref, x3_ref, x4_ref)):
        topk_w, probs, topk_i, logits = _gate_half(x_ref[0, 0], w)
        topk_w_ref[h] = topk_w
        probs_ref[h] = probs
        topk_i_ref[h] = topk_i
        logits_ref[h] = logits


@functools.partial(jax.jit, static_argnames=())
def kernel(x, W):
    n_tokens, d_model = x.shape
    n_experts = W.shape[0]
    half = n_tokens // _NS
    half_blocks = half // _BT
    xr = x.reshape(_NS, half_blocks, _BT, d_model)
    grid = (half_blocks,)
    out = pl.pallas_call(
        _gate_kernel,
        grid=grid,
        in_specs=[
            pl.BlockSpec((1, 1, _BT, d_model), lambda i: (0, i, 0, 0)),
            pl.BlockSpec((1, 1, _BT, d_model), lambda i: (1, i, 0, 0)),
            pl.BlockSpec((1, 1, _BT, d_model), lambda i: (2, i, 0, 0)),
            pl.BlockSpec((1, 1, _BT, d_model), lambda i: (3, i, 0, 0)),
            pl.BlockSpec((n_experts, d_model), lambda i: (0, 0)),
        ],
        out_specs=[
            pl.BlockSpec((_NS, _BT, _TOP_K), lambda i: (0, i, 0)),
            pl.BlockSpec((_NS, _BT, _NUM_EXPERTS), lambda i: (0, i, 0)),
            pl.BlockSpec((_NS, _BT, _TOP_K), lambda i: (0, i, 0)),
            pl.BlockSpec((_NS, _BT, _NUM_EXPERTS), lambda i: (0, i, 0)),
        ],
        out_shape=[
            jax.ShapeDtypeStruct((_NS, half, _TOP_K), jnp.float32),
            jax.ShapeDtypeStruct((_NS, half, _NUM_EXPERTS), jnp.float32),
            jax.ShapeDtypeStruct((_NS, half, _TOP_K), jnp.int32),
            jax.ShapeDtypeStruct((_NS, half, _NUM_EXPERTS), jnp.float32),
        ],
        compiler_params=pltpu.CompilerParams(
            dimension_semantics=("arbitrary",),
        ),
    )(xr, xr, xr, xr, W)
    topk_w, probs, topk_i, logits = out
    return (
        topk_w.reshape(n_tokens, _TOP_K),
        probs.reshape(n_tokens, _NUM_EXPERTS),
        topk_i.reshape(n_tokens, _TOP_K),
        logits.reshape(n_tokens, _NUM_EXPERTS),
    )


# speedup vs baseline: 1.0064x; 1.0064x over previous
import functools

import jax
import jax.numpy as jnp
from jax.experimental import pallas as pl
from jax.experimental.pallas import tpu as pltpu

_D_MODEL = 4096
_NUM_EXPERTS = 64
_TOP_K = 8
_BT = 512


def _gate_half(x_blk, w):
    logits_t = jax.lax.dot_general(
        w, x_blk,
        dimension_numbers=(((1,), (1,)), ((), ())),
        preferred_element_type=jnp.float32,
    )
    m = jnp.max(logits_t, axis=0, keepdims=True)
    e = jnp.exp(logits_t - m)
    s = jnp.sum(e, axis=0, keepdims=True)
    probs_t = e / s
    psum = jnp.sum(probs_t, axis=0, keepdims=True)

    eiota = jax.lax.broadcasted_iota(jnp.int32, probs_t.shape, 0).astype(
        jnp.float32
    )
    cur = probs_t
    mxs = []
    idxs = []
    for k in range(_TOP_K):
        mx = jnp.max(cur, axis=0, keepdims=True)
        idxf = jnp.min(
            jnp.where(cur == mx, eiota, float(_NUM_EXPERTS)),
            axis=0, keepdims=True,
        )
        mxs.append(mx / psum)
        idxs.append(idxf)
        if k + 1 < _TOP_K:
            cur = jnp.where(eiota == idxf, -1.0, cur)
    topk_w_t = jnp.concatenate(mxs, axis=0)
    topk_i_t = jnp.concatenate(idxs, axis=0)
    return (topk_w_t.T, probs_t.T, topk_i_t.T.astype(jnp.int32), logits_t.T)


def _gate_kernel(x1_ref, x2_ref, w_ref, topk_w_ref, probs_ref, topk_i_ref,
                 logits_ref):
    w = w_ref[...]
    for h, x_ref in enumerate((x1_ref, x2_ref)):
        topk_w, probs, topk_i, logits = _gate_half(x_ref[0, 0], w)
        topk_w_ref[h] = topk_w
        probs_ref[h] = probs
        topk_i_ref[h] = topk_i
        logits_ref[h] = logits


@functools.partial(jax.jit, static_argnames=())
def kernel(x, W):
    n_tokens, d_model = x.shape
    n_experts = W.shape[0]
    half = n_tokens // 2
    half_blocks = half // _BT
    xr = x.reshape(2, half_blocks, _BT, d_model)
    grid = (half_blocks,)
    out = pl.pallas_call(
        _gate_kernel,
        grid=grid,
        in_specs=[
            pl.BlockSpec((1, 1, _BT, d_model), lambda i: (0, i, 0, 0)),
            pl.BlockSpec((1, 1, _BT, d_model), lambda i: (1, i, 0, 0)),
            pl.BlockSpec((n_experts, d_model), lambda i: (0, 0)),
        ],
        out_specs=[
            pl.BlockSpec((2, _BT, _TOP_K), lambda i: (0, i, 0)),
            pl.BlockSpec((2, _BT, _NUM_EXPERTS), lambda i: (0, i, 0)),
            pl.BlockSpec((2, _BT, _TOP_K), lambda i: (0, i, 0)),
            pl.BlockSpec((2, _BT, _NUM_EXPERTS), lambda i: (0, i, 0)),
        ],
        out_shape=[
            jax.ShapeDtypeStruct((2, half, _TOP_K), jnp.float32),
            jax.ShapeDtypeStruct((2, half, _NUM_EXPERTS), jnp.float32),
            jax.ShapeDtypeStruct((2, half, _TOP_K), jnp.int32),
            jax.ShapeDtypeStruct((2, half, _NUM_EXPERTS), jnp.float32),
        ],
        compiler_params=pltpu.CompilerParams(
            dimension_semantics=("arbitrary",),
        ),
    )(xr, xr, W)
    topk_w, probs, topk_i, logits = out
    return (
        topk_w.reshape(n_tokens, _TOP_K),
        probs.reshape(n_tokens, _NUM_EXPERTS),
        topk_i.reshape(n_tokens, _TOP_K),
        logits.reshape(n_tokens, _NUM_EXPERTS),
    )
